# baseline (speedup 1.0000x reference)
import functools

import jax
import jax.numpy as jnp
from jax import lax
from jax.experimental import pallas as pl
from jax.experimental.pallas import tpu as pltpu
from jax.experimental.pallas import tpu_sc as plsc

TIME = 4096
FREQ = 513
CHUNK = 20
N_WIN = TIME - CHUNK
JB = 256
NJ = (N_WIN + JB - 1) // JB
PADW = TIME + 128


def _tc_body(mt_hbm, x_ref, mscr, sem0):
    jb = pl.program_id(0)

    @pl.when(jb == 0)
    def _():
        pltpu.make_async_copy(mt_hbm, mscr.at[:, pl.ds(0, TIME)], sem0).start()
        pltpu.make_async_copy(mt_hbm, mscr.at[:, pl.ds(0, TIME)], sem0).wait()

    base = pl.multiple_of(jb * JB, 128)
    w = mscr[:, pl.ds(base, JB + 128)]
    for c in range(CHUNK):
        x_ref[c, :, :] = w[:, c:c + JB]


_tc_call = pl.pallas_call(
    _tc_body,
    grid=(NJ,),
    in_specs=[pl.BlockSpec(memory_space=pl.ANY)],
    out_specs=[pl.BlockSpec((CHUNK, FREQ, JB), lambda j: (0, 0, j))],
    out_shape=[jax.ShapeDtypeStruct((CHUNK, FREQ, N_WIN), jnp.float32)],
    scratch_shapes=[
        pltpu.VMEM((FREQ, PADW), jnp.float32),
        pltpu.SemaphoreType.DMA,
    ],
    compiler_params=pltpu.CompilerParams(vmem_limit_bytes=58 * 1024 * 1024),
)

_mesh = plsc.VectorSubcoreMesh(core_axis_name="c", subcore_axis_name="s")


@functools.partial(
    pl.kernel,
    mesh=_mesh,
    out_type=jax.ShapeDtypeStruct((FREQ, N_WIN), jnp.float32),
    scratch_types=[
        pltpu.VMEM((9, TIME), jnp.float32),
        pltpu.VMEM((9, N_WIN), jnp.float32),
    ],
    compiler_params=pltpu.CompilerParams(needs_layout_passes=False),
)
def _sc_y(vt_ref, y_ref, in_s, out_s):
    w = lax.axis_index("s") * 2 + lax.axis_index("c")
    lanes = lax.iota(jnp.int32, 16)

    def assemble(nrows):
        for s in range(nrows):
            rows = jnp.full((16,), s, jnp.int32)

            @plsc.parallel_loop(0, N_WIN - 12, 16)
            def _(l0):
                cols = l0 + CHUNK + lanes
                out_s[s, pl.ds(l0, 16)] = plsc.load_gather(in_s, [rows, cols])

            tail = N_WIN - 16
            out_s[s, pl.ds(tail, 16)] = plsc.load_gather(
                in_s, [rows, tail + CHUNK + lanes])

    def do_tile(t):
        r0 = pl.multiple_of(8 * t, 8)
        pltpu.sync_copy(vt_ref.at[pl.ds(r0, 8), :], in_s.at[pl.ds(0, 8), :])
        assemble(8)
        pltpu.sync_copy(out_s.at[pl.ds(0, 8), :], y_ref.at[pl.ds(r0, 8), :])

    do_tile(w)

    @pl.when(w < 31)
    def _():
        do_tile(w + 32)

    @pl.when(w == 31)
    def _():
        pltpu.sync_copy(vt_ref.at[pl.ds(504, 9), :], in_s)
        assemble(9)
        pltpu.sync_copy(out_s, y_ref.at[pl.ds(504, 9), :])


def kernel(mixed_mag, vocal_mag):
    mt = mixed_mag.T
    vt = vocal_mag.T
    (x_alt,) = _tc_call(mt)
    y_alt = _sc_y(vt)
    return x_alt.transpose(2, 1, 0), y_alt.T
